# Initial kernel scaffold; baseline (speedup 1.0000x reference)
import jax
import jax.numpy as jnp
from jax.experimental import pallas as pl


def kernel(base_mask, pert_indices, incidence, incidence_mask):
    raise NotImplementedError("write your pallas kernel here")


# speedup vs baseline: 54.5418x; 54.5418x over previous
import functools

import jax
import jax.numpy as jnp
from jax import lax
from jax.experimental import pallas as pl
from jax.experimental.pallas import tpu as pltpu
from jax.experimental.pallas import tpu_sc as plsc

NC = 2
NS = 16
L = 16
NW = NC * NS

PP = 8
FAR = 1 << 30


def _make_sc_kernel(E, B, P, G, D, CH=25000, NBUF=3, interpret=False):
    WPS = NW // B
    HALF = E // WPS
    assert HALF % CH == 0 and CH % 8 == 0
    NCHUNK = HALF // CH
    NPOS = P * D
    NT = -(-NPOS // L)
    NPF = NT * L
    NR = -(-NPOS // 128)
    W = NBUF - 1

    mesh = plsc.VectorSubcoreMesh(core_axis_name="c", subcore_axis_name="s",
                                  num_cores=NC, num_subcores=NS)

    @functools.partial(
        pl.kernel,
        out_type=jax.ShapeDtypeStruct((B * E,), jnp.float32),
        mesh=mesh,
        interpret=interpret,
        compiler_params=pltpu.CompilerParams(needs_layout_passes=False),
        scratch_types=[
            pltpu.VMEM((B, PP), jnp.int32),
            pltpu.VMEM((NR, 128), jnp.int32),
            pltpu.VMEM((NR, 128), jnp.int32),
            pltpu.VMEM((NPF,), jnp.int32),
            [pltpu.VMEM((CH + L,), jnp.float32) for _ in range(NBUF)],
            pltpu.VMEM_SHARED((HALF,), jnp.float32),
            [pltpu.SemaphoreType.DMA for _ in range(NBUF)],
            [pltpu.SemaphoreType.DMA for _ in range(NBUF)],
            pltpu.SemaphoreType.DMA,
            pltpu.SemaphoreType.DMA,
        ],
    )
    def body(base_hbm, pert_hbm, inc_hbm, out_hbm,
             pert_v, adr_v, val_v, pos_v, bufs, shared, sin, sout, sem, sst):
        wid = lax.axis_index("s") * NC + lax.axis_index("c")
        b = wid // WPS
        h0 = (wid % WPS) * HALF
        sub = lax.axis_index("s")

        SLICE = HALF // NS
        HS = SLICE // 2
        assert HS <= CH
        dleg = [pltpu.async_copy(
                    base_hbm.at[pl.ds(h0 + sub * SLICE + j * HS, HS)],
                    bufs[j].at[pl.ds(0, HS)], sin[j])
                for j in range(2)]
        dstage = []
        for j in range(2):
            dleg[j].wait()
            dstage.append(pltpu.async_copy(
                bufs[j].at[pl.ds(0, HS)],
                shared.at[pl.ds(sub * SLICE + j * HS, HS)], sst))

        def start_in(c):
            k = c % NBUF
            if c % 2 == 0:
                src = shared.at[pl.ds(c * CH, CH)]
            else:
                src = base_hbm.at[pl.ds(h0 + c * CH, CH)]
            return pltpu.async_copy(src, bufs[k].at[pl.ds(0, CH)], sin[k])

        def start_out(c):
            k = c % NBUF
            return pltpu.async_copy(
                bufs[k].at[pl.ds(0, CH)],
                out_hbm.at[pl.ds(b * E + h0 + c * CH, CH)], sout[k])

        pltpu.sync_copy(pert_hbm, pert_v)
        dout = {}

        iota = lax.iota(jnp.int32, L)
        b_vec = jnp.full((L,), b, jnp.int32)
        for q in range(NR * 128 // L):
            f = jnp.minimum(q * L + iota, NPOS - 1)
            slot = f // P
            pcol = f - slot * P
            gene = plsc.load_gather(pert_v, [b_vec, pcol])
            adr_v[q * L // 128, pl.ds(q * L % 128, L)] = slot * G + gene
        dgather = [
            pltpu.async_copy(inc_hbm.at[adr_v.at[r]], val_v.at[r], sem)
            for r in range(NR)
        ]
        for dsc in dgather:
            dsc.wait()
        for t in range(NT):
            q0 = t * L
            pos = val_v[q0 // 128, pl.ds(q0 % 128, L)]
            pos_v[pl.ds(t * L, L)] = jnp.where(pos >= 0, pos - h0, FAR)

        for dsc in dstage:
            dsc.wait()
        plsc.subcore_barrier()
        din = {j: start_in(j) for j in range(min(W, NCHUNK))}

        zeros = jnp.zeros((L,), jnp.float32)

        for c in range(NCHUNK):
            nxt = c + W
            if nxt < NCHUNK:
                if nxt >= NBUF:
                    dout[nxt - NBUF].wait()
                din[nxt] = start_in(nxt)
            din[c].wait()
            for t in range(NT):
                li = pos_v[pl.ds(t * L, L)] - c * CH
                ok = (li >= 0) & (li < CH)
                idx = jnp.where(ok, li, CH)
                plsc.store_scatter(bufs[c % NBUF], [idx], zeros)
            dout[c] = start_out(c)
        for c in range(max(0, NCHUNK - NBUF), NCHUNK):
            dout[c].wait()

    return body


def kernel(base_mask, pert_indices, incidence, incidence_mask):
    E = base_mask.shape[0]
    B, P = pert_indices.shape
    G, D = incidence.shape
    pert = pert_indices.astype(jnp.int32)
    pert8 = jnp.concatenate(
        [pert, jnp.broadcast_to(pert[:, :1], (B, PP - P))], axis=1)
    inc_flat = jnp.swapaxes(incidence.astype(jnp.int32), 0, 1).reshape(-1)
    sc = _make_sc_kernel(E, B, P, G, D)
    out = sc(base_mask.astype(jnp.float32), pert8, inc_flat)
    return out


# speedup vs baseline: 73.2240x; 1.3425x over previous
import functools

import jax
import jax.numpy as jnp
from jax import lax
from jax.experimental import pallas as pl
from jax.experimental.pallas import tpu as pltpu
from jax.experimental.pallas import tpu_sc as plsc

NC = 2
NS = 16
L = 16
NW = NC * NS

PP = 8
FAR = 1 << 30


def _make_sc_kernel(E, B, P, G, D, CH=25000, NBUF=3, interpret=False):
    WPS = NW // B
    HALF = E // WPS
    assert HALF % CH == 0 and CH % 8 == 0
    NCHUNK = HALF // CH
    NPOS = P * D
    NT = -(-NPOS // L)
    NPF = NT * L
    NR = -(-NPOS // 128)
    W = NBUF - 1

    mesh = plsc.VectorSubcoreMesh(core_axis_name="c", subcore_axis_name="s",
                                  num_cores=NC, num_subcores=NS)

    @functools.partial(
        pl.kernel,
        out_type=jax.ShapeDtypeStruct((B * E,), jnp.float32),
        mesh=mesh,
        interpret=interpret,
        compiler_params=pltpu.CompilerParams(needs_layout_passes=False),
        scratch_types=[
            pltpu.VMEM((B, PP), jnp.int32),
            pltpu.VMEM((NR, 128), jnp.int32),
            pltpu.VMEM((NR, 128), jnp.int32),
            pltpu.VMEM((NPF,), jnp.int32),
            [pltpu.VMEM((CH + L,), jnp.float32) for _ in range(NBUF)],
            pltpu.VMEM_SHARED((HALF,), jnp.float32),
            [pltpu.SemaphoreType.DMA for _ in range(NBUF)],
            [pltpu.SemaphoreType.DMA for _ in range(NBUF)],
            pltpu.SemaphoreType.DMA,
            pltpu.SemaphoreType.DMA,
        ],
    )
    def body(base_hbm, pert_hbm, inc_hbm, out_hbm,
             pert_v, adr_v, val_v, pos_v, bufs, shared, sin, sout, sem, sst):
        wid = lax.axis_index("s") * NC + lax.axis_index("c")
        b = wid // WPS
        h0 = (wid % WPS) * HALF
        sub = lax.axis_index("s")

        SLICE = HALF // NS
        HS = SLICE // 2
        assert HS <= CH
        dleg = [pltpu.async_copy(
                    base_hbm.at[pl.ds(h0 + sub * SLICE + j * HS, HS)],
                    bufs[j].at[pl.ds(0, HS)], sin[j])
                for j in range(2)]
        dstage = []
        for j in range(2):
            dleg[j].wait()
            dstage.append(pltpu.async_copy(
                bufs[j].at[pl.ds(0, HS)],
                shared.at[pl.ds(sub * SLICE + j * HS, HS)], sst))

        def start_in(c):
            k = c % NBUF
            return pltpu.async_copy(
                shared.at[pl.ds(c * CH, CH)],
                bufs[k].at[pl.ds(0, CH)], sin[k])

        def start_out(c):
            k = c % NBUF
            return pltpu.async_copy(
                bufs[k].at[pl.ds(0, CH)],
                out_hbm.at[pl.ds(b * E + h0 + c * CH, CH)], sout[k])

        pltpu.sync_copy(pert_hbm, pert_v)
        dout = {}

        iota = lax.iota(jnp.int32, L)
        b_vec = jnp.full((L,), b, jnp.int32)
        for q in range(NR * 128 // L):
            f = jnp.minimum(q * L + iota, NPOS - 1)
            slot = f // P
            pcol = f - slot * P
            gene = plsc.load_gather(pert_v, [b_vec, pcol])
            adr_v[q * L // 128, pl.ds(q * L % 128, L)] = slot * G + gene
        dgather = [
            pltpu.async_copy(inc_hbm.at[adr_v.at[r]], val_v.at[r], sem)
            for r in range(NR)
        ]
        for dsc in dgather:
            dsc.wait()
        for t in range(NT):
            q0 = t * L
            pos = val_v[q0 // 128, pl.ds(q0 % 128, L)]
            pos_v[pl.ds(t * L, L)] = jnp.where(pos >= 0, pos - h0, FAR)

        for dsc in dstage:
            dsc.wait()
        plsc.subcore_barrier()
        din = {j: start_in(j) for j in range(min(W, NCHUNK))}

        zeros = jnp.zeros((L,), jnp.float32)

        for c in range(NCHUNK):
            nxt = c + W
            if nxt < NCHUNK:
                if nxt >= NBUF:
                    dout[nxt - NBUF].wait()
                din[nxt] = start_in(nxt)
            din[c].wait()
            for t in range(NT):
                li = pos_v[pl.ds(t * L, L)] - c * CH
                ok = (li >= 0) & (li < CH)
                idx = jnp.where(ok, li, CH)
                plsc.store_scatter(bufs[c % NBUF], [idx], zeros)
            dout[c] = start_out(c)
        for c in range(max(0, NCHUNK - NBUF), NCHUNK):
            dout[c].wait()

    return body


def kernel(base_mask, pert_indices, incidence, incidence_mask):
    E = base_mask.shape[0]
    B, P = pert_indices.shape
    G, D = incidence.shape
    pert = pert_indices.astype(jnp.int32)
    pert8 = jnp.concatenate(
        [pert, jnp.broadcast_to(pert[:, :1], (B, PP - P))], axis=1)
    inc_flat = jnp.swapaxes(incidence.astype(jnp.int32), 0, 1).reshape(-1)
    sc = _make_sc_kernel(E, B, P, G, D)
    out = sc(base_mask.astype(jnp.float32), pert8, inc_flat)
    return out
